# baseline (speedup 1.0000x reference)
import functools

import jax
import jax.numpy as jnp
from jax import lax
from jax.experimental import pallas as pl
from jax.experimental.pallas import tpu as pltpu
from jax.experimental.pallas import tpu_sc as plsc

B_TOK = 4096
S = 200
D = 64
NC = 2
NS = 16
NW = NC * NS
BW = B_TOK // NW
TPW = BW * S
L = 16


def _build_gather():
    mesh = plsc.VectorSubcoreMesh(core_axis_name="c", subcore_axis_name="s")

    @functools.partial(
        pl.kernel,
        mesh=mesh,
        out_type=jax.ShapeDtypeStruct((S, 8, NW, 8, 128), jnp.float32),
        compiler_params=pltpu.CompilerParams(use_tc_tiling_on_sc=False,
                                             needs_layout_passes=False),
        scratch_types=[
            pltpu.VMEM((TPW,), jnp.int32),
            pltpu.VMEM((2, BW), jnp.int32),
            pltpu.VMEM((2, BW, D), jnp.float32),
            pltpu.VMEM((2, D, 128), jnp.float32),
        ] + [pltpu.SemaphoreType.DMA] * 4,
    )
    def gather_kernel(table_hbm, idx_hbm, out_hbm, idx_v, list_v, g_v, st_v,
                      gsem0, gsem1, osem0, osem1):
        wid = lax.axis_index("s") * NC + lax.axis_index("c")
        gsem = (gsem0, gsem1)
        osem = (osem0, osem1)
        pltpu.sync_copy(idx_hbm.at[pl.ds(wid * TPW, TPW)], idx_v)

        jv = lax.iota(jnp.int32, L)

        def fire(s, p):
            for k in range(BW // L):
                pos = (jv + (L * k)) * S + s
                tv = plsc.load_gather(idx_v, [pos])
                list_v[p, pl.ds(L * k, L)] = tv
            pltpu.async_copy(table_hbm.at[list_v.at[p]], g_v.at[p], gsem[p])

        def wait_g(p):
            pltpu.make_async_copy(
                table_hbm.at[list_v.at[p]], g_v.at[p], gsem[p]).wait()

        def transpose(p):
            g2 = g_v.at[p]

            @pl.loop(0, D, step=8)
            def _(c0):
                for cc in range(8):
                    c = c0 + cc
                    cv = jnp.full((L,), 0, jnp.int32) + c
                    for k in range(BW // L):
                        rv = jv + (L * k)
                        v = plsc.load_gather(g2, [rv, cv])
                        st_v[p, c, pl.ds(L * k, L)] = v

        def fire_out(s, p):
            for dt in range(8):
                pltpu.async_copy(st_v.at[p, pl.ds(8 * dt, 8)],
                                 out_hbm.at[s, dt, wid], osem[p])

        def wait_out(p):
            for dt in range(8):
                pltpu.make_async_copy(out_hbm.at[0, dt, 0],
                                      st_v.at[p, pl.ds(8 * dt, 8)],
                                      osem[p]).wait()

        fire(0, 0)

        @pl.loop(0, S, step=2)
        def _(s0):
            for b in range(2):
                s = s0 + b
                p = b

                @pl.when(s < S - 1)
                def _():
                    fire(s + 1, 1 - p)

                wait_g(p)

                @pl.when(s >= 2)
                def _():
                    wait_out(p)

                transpose(p)
                fire_out(s, p)

        wait_out(0)
        wait_out(1)

    return gather_kernel


def kernel(x, weight):
    idx = x.reshape(B_TOK * S).astype(jnp.int32)
    o5 = _build_gather()(weight, idx)
    out = o5.transpose(0, 1, 3, 2, 4).reshape(S, D, B_TOK)
    return out.transpose(2, 0, 1)


# speedup vs baseline: 1.7871x; 1.7871x over previous
import functools

import jax
import jax.numpy as jnp
from jax import lax
from jax.experimental import pallas as pl
from jax.experimental.pallas import tpu as pltpu
from jax.experimental.pallas import tpu_sc as plsc

B_TOK = 4096
S = 200
D = 64
NC = 2
NS = 16
NW = NC * NS
BW = B_TOK // NW
TPW = BW * S
L = 16


def _build_gather():
    mesh = plsc.VectorSubcoreMesh(core_axis_name="c", subcore_axis_name="s")

    @functools.partial(
        pl.kernel,
        mesh=mesh,
        out_type=jax.ShapeDtypeStruct((S, 8, NW, 8, 128), jnp.float32),
        compiler_params=pltpu.CompilerParams(use_tc_tiling_on_sc=False,
                                             needs_layout_passes=False),
        scratch_types=[
            pltpu.VMEM((BW, S + 1), jnp.int32),
            pltpu.VMEM((2, BW), jnp.int32),
            pltpu.VMEM((2, BW, D), jnp.float32),
            pltpu.VMEM((2, D, 128), jnp.float32),
        ] + [pltpu.SemaphoreType.DMA] * 4,
    )
    def gather_kernel(table_hbm, idx_hbm, out_hbm, idx_v, list_v, g_v, st_v,
                      gsem0, gsem1, osem0, osem1):
        wid = lax.axis_index("s") * NC + lax.axis_index("c")
        gsem = (gsem0, gsem1)
        osem = (osem0, osem1)
        pltpu.sync_copy(idx_hbm.at[pl.ds(wid * BW, BW)],
                        idx_v.at[pl.ds(0, BW), pl.ds(0, S)])

        jv = lax.iota(jnp.int32, L)

        def fire(s, p):
            sv = jnp.full((L,), 0, jnp.int32) + s
            for k in range(BW // L):
                rv = jv + (L * k)
                tv = plsc.load_gather(idx_v, [rv, sv])
                list_v[p, pl.ds(L * k, L)] = tv
            pltpu.async_copy(table_hbm.at[list_v.at[p]], g_v.at[p], gsem[p])

        def wait_g(p):
            pltpu.make_async_copy(
                table_hbm.at[list_v.at[p]], g_v.at[p], gsem[p]).wait()

        def transpose(p):
            g2 = g_v.at[p]
            st2 = st_v.at[p]

            @pl.loop(0, D, step=8)
            def _(c0):
                for cc in range(8):
                    dv = (jv + (c0 + cc)) & (D - 1)
                    for k in range(BW // L):
                        rv = jv + (L * k)
                        v = plsc.load_gather(g2, [rv, dv])
                        plsc.store_scatter(st2, [dv, rv], v)

        def fire_out(s, p):
            for dt in range(8):
                pltpu.async_copy(st_v.at[p, pl.ds(8 * dt, 8)],
                                 out_hbm.at[s, dt, wid], osem[p])

        def wait_out(p):
            for dt in range(8):
                pltpu.make_async_copy(out_hbm.at[0, dt, 0],
                                      st_v.at[p, pl.ds(8 * dt, 8)],
                                      osem[p]).wait()

        fire(0, 0)

        @pl.loop(0, S, step=2)
        def _(s0):
            for b in range(2):
                s = s0 + b
                p = b

                @pl.when(s < S - 1)
                def _():
                    fire(s + 1, 1 - p)

                wait_g(p)

                @pl.when(s >= 2)
                def _():
                    wait_out(p)

                transpose(p)
                fire_out(s, p)

        wait_out(0)
        wait_out(1)

    return gather_kernel


def kernel(x, weight):
    o5 = _build_gather()(weight, x.astype(jnp.int32))
    out = o5.transpose(0, 1, 3, 2, 4).reshape(S, D, B_TOK)
    return out.transpose(2, 0, 1)


# speedup vs baseline: 2.1957x; 1.2286x over previous
import functools

import jax
import jax.numpy as jnp
from jax import lax
from jax.experimental import pallas as pl
from jax.experimental.pallas import tpu as pltpu
from jax.experimental.pallas import tpu_sc as plsc

B_TOK = 4096
S = 200
D = 64
NC = 2
NS = 16
NW = NC * NS
BW = B_TOK // NW
TPW = BW * S
L = 16


def _build_gather():
    mesh = plsc.VectorSubcoreMesh(core_axis_name="c", subcore_axis_name="s")

    @functools.partial(
        pl.kernel,
        mesh=mesh,
        out_type=jax.ShapeDtypeStruct((S, 8, NW, 8, 128), jnp.float32),
        compiler_params=pltpu.CompilerParams(use_tc_tiling_on_sc=False,
                                             needs_layout_passes=False),
        scratch_types=[
            pltpu.VMEM((BW, S + 1), jnp.int32),
            pltpu.VMEM((2, BW), jnp.int32),
            pltpu.VMEM((2, BW, D), jnp.float32),
            pltpu.VMEM((2, D, 128), jnp.float32),
        ] + [pltpu.SemaphoreType.DMA] * 4,
    )
    def gather_kernel(table_hbm, idx_hbm, out_hbm, idx_v, list_v, g_v, st_v,
                      gsem0, gsem1, osem0, osem1):
        wid = lax.axis_index("s") * NC + lax.axis_index("c")
        gsem = (gsem0, gsem1)
        osem = (osem0, osem1)
        pltpu.sync_copy(idx_hbm.at[pl.ds(wid * BW, BW)],
                        idx_v.at[pl.ds(0, BW), pl.ds(0, S)])

        jv = lax.iota(jnp.int32, L)

        def fire(s, p):
            sv = jnp.full((L,), 0, jnp.int32) + s
            for k in range(BW // L):
                rv = jv + (L * k)
                tv = plsc.load_gather(idx_v, [rv, sv])
                list_v[p, pl.ds(L * k, L)] = tv
            pltpu.async_copy(table_hbm.at[list_v.at[p]], g_v.at[p], gsem[p])

        def wait_g(p):
            pltpu.make_async_copy(
                table_hbm.at[list_v.at[p]], g_v.at[p], gsem[p]).wait()

        def transpose(p):
            g2 = g_v.at[p]
            st2 = st_v.at[p]

            @plsc.parallel_loop(0, D, step=8, unroll=2)
            def _(c0):
                for cc in range(8):
                    dv = (jv + (c0 + cc)) & (D - 1)
                    for k in range(BW // L):
                        rv = jv + (L * k)
                        v = plsc.load_gather(g2, [rv, dv])
                        plsc.store_scatter(st2, [dv, rv], v)

        def fire_out(s, p):
            for dt in range(8):
                pltpu.async_copy(st_v.at[p, pl.ds(8 * dt, 8)],
                                 out_hbm.at[s, dt, wid], osem[p])

        def wait_out(p):
            for dt in range(8):
                pltpu.make_async_copy(out_hbm.at[0, dt, 0],
                                      st_v.at[p, pl.ds(8 * dt, 8)],
                                      osem[p]).wait()

        fire(0, 0)

        @pl.loop(0, S, step=2)
        def _(s0):
            for b in range(2):
                s = s0 + b
                p = b

                @pl.when(s < S - 1)
                def _():
                    fire(s + 1, 1 - p)

                wait_g(p)

                @pl.when(s >= 2)
                def _():
                    wait_out(p)

                transpose(p)
                fire_out(s, p)

        wait_out(0)
        wait_out(1)

    return gather_kernel


def kernel(x, weight):
    o5 = _build_gather()(weight, x.astype(jnp.int32))
    out = o5.transpose(0, 1, 3, 2, 4).reshape(S, D, B_TOK)
    return out.transpose(2, 0, 1)


# speedup vs baseline: 2.3898x; 1.0884x over previous
import functools

import jax
import jax.numpy as jnp
from jax import lax
from jax.experimental import pallas as pl
from jax.experimental.pallas import tpu as pltpu
from jax.experimental.pallas import tpu_sc as plsc

B_TOK = 4096
S = 200
D = 64
NC = 2
NS = 16
NW = NC * NS
BW = B_TOK // NW
TPW = BW * S
L = 16


def _build_gather():
    mesh = plsc.VectorSubcoreMesh(core_axis_name="c", subcore_axis_name="s")

    @functools.partial(
        pl.kernel,
        mesh=mesh,
        out_type=jax.ShapeDtypeStruct((S, 8, NW, 8, 128), jnp.float32),
        compiler_params=pltpu.CompilerParams(use_tc_tiling_on_sc=False,
                                             needs_layout_passes=False),
        scratch_types=[
            pltpu.VMEM((BW, S + 1), jnp.int32),
            pltpu.VMEM((2, BW), jnp.int32),
            pltpu.VMEM((2, BW, D), jnp.float32),
            pltpu.VMEM((2, D, 128), jnp.float32),
        ] + [pltpu.SemaphoreType.DMA] * 4,
    )
    def gather_kernel(table_hbm, idx_hbm, out_hbm, idx_v, list_v, g_v, st_v,
                      gsem0, gsem1, osem0, osem1):
        wid = lax.axis_index("s") * NC + lax.axis_index("c")
        gsem = (gsem0, gsem1)
        osem = (osem0, osem1)
        pltpu.sync_copy(idx_hbm.at[pl.ds(wid * BW, BW)],
                        idx_v.at[pl.ds(0, BW), pl.ds(0, S)])

        jv = lax.iota(jnp.int32, L)

        def fire(s, p):
            sv = jnp.full((L,), 0, jnp.int32) + s
            for k in range(BW // L):
                rv = jv + (L * k)
                tv = plsc.load_gather(idx_v, [rv, sv])
                list_v[p, pl.ds(L * k, L)] = tv
            pltpu.async_copy(table_hbm.at[list_v.at[p]], g_v.at[p], gsem[p])

        def wait_g(p):
            pltpu.make_async_copy(
                table_hbm.at[list_v.at[p]], g_v.at[p], gsem[p]).wait()

        def transpose(p):
            g2 = g_v.at[p]
            st2 = st_v.at[p]

            @plsc.parallel_loop(0, D, step=8, unroll=4)
            def _(c0):
                for cc in range(8):
                    dv = (jv + (c0 + cc)) & (D - 1)
                    for k in range(BW // L):
                        rv = jv + (L * k)
                        v = plsc.load_gather(g2, [rv, dv])
                        plsc.store_scatter(st2, [dv, rv], v)

        def fire_out(s, p):
            for dt in range(8):
                pltpu.async_copy(st_v.at[p, pl.ds(8 * dt, 8)],
                                 out_hbm.at[s, dt, wid], osem[p])

        def wait_out(p):
            for dt in range(8):
                pltpu.make_async_copy(out_hbm.at[0, dt, 0],
                                      st_v.at[p, pl.ds(8 * dt, 8)],
                                      osem[p]).wait()

        fire(0, 0)

        @pl.loop(0, S, step=2)
        def _(s0):
            for b in range(2):
                s = s0 + b
                p = b

                @pl.when(s < S - 1)
                def _():
                    fire(s + 1, 1 - p)

                wait_g(p)

                @pl.when(s >= 2)
                def _():
                    wait_out(p)

                transpose(p)
                fire_out(s, p)

        wait_out(0)
        wait_out(1)

    return gather_kernel


def kernel(x, weight):
    o5 = _build_gather()(weight, x.astype(jnp.int32))
    out = o5.transpose(0, 1, 3, 2, 4).reshape(S, D, B_TOK)
    return out.transpose(2, 0, 1)


# speedup vs baseline: 2.7638x; 1.1565x over previous
import functools

import jax
import jax.numpy as jnp
from jax import lax
from jax.experimental import pallas as pl
from jax.experimental.pallas import tpu as pltpu
from jax.experimental.pallas import tpu_sc as plsc

B_TOK = 4096
S = 200
D = 64
NC = 2
NS = 16
NW = NC * NS
BW = B_TOK // NW
TPW = BW * S
L = 16


SLABS = 125000
SPW = 3906
RND = 48
NR_FULL = 81
TAIL = 18
EXTRA = 8


def _build_depad():
    mesh = plsc.VectorSubcoreMesh(core_axis_name="c", subcore_axis_name="s")

    @functools.partial(
        pl.kernel,
        mesh=mesh,
        out_type=jax.ShapeDtypeStruct((SLABS * 8 * D,), jnp.float32),
        compiler_params=pltpu.CompilerParams(use_tc_tiling_on_sc=True),
        scratch_types=[
            pltpu.VMEM((2, RND, 8, D), jnp.float32),
            pltpu.VMEM((RND * 8 * D,), jnp.float32),
        ] + [pltpu.SemaphoreType.DMA] * 3,
    )
    def depad_kernel(w3_hbm, out_hbm, vb, vdrain, isem0, isem1, osem):
        wid = lax.axis_index("s") * NC + lax.axis_index("c")
        base = wid * SPW
        isem = (isem0, isem1)

        def fire_in(r, b, cnt):
            pltpu.async_copy(w3_hbm.at[pl.ds(base + r * RND, cnt)],
                             vb.at[b, pl.ds(0, cnt)], isem[b])

        def flush(r, b, cnt):
            pltpu.make_async_copy(w3_hbm.at[pl.ds(base + r * RND, cnt)],
                                  vb.at[b, pl.ds(0, cnt)], isem[b]).wait()
            for i in range(cnt):
                for rr in range(8):
                    pltpu.async_copy(
                        vb.at[b, i, rr],
                        out_hbm.at[pl.ds((base + r * RND + i) * 512 + rr * D,
                                         D)],
                        osem)
            n = cnt * 8 * D
            pltpu.make_async_copy(out_hbm.at[pl.ds(0, n)],
                                  vdrain.at[pl.ds(0, n)], osem).wait()

        fire_in(0, 0, RND)

        @pl.loop(0, NR_FULL - 1, step=2)
        def _(r0):
            for b in range(2):
                r = r0 + b
                fire_in(r + 1, 1 - b, RND)
                flush(r, b, RND)

        pltpu.async_copy(w3_hbm.at[pl.ds(base + NR_FULL * RND, TAIL)],
                         vb.at[1, pl.ds(0, TAIL)], isem1)
        flush(NR_FULL - 1, 0, RND)
        pltpu.make_async_copy(w3_hbm.at[pl.ds(base + NR_FULL * RND, TAIL)],
                              vb.at[1, pl.ds(0, TAIL)], isem1).wait()
        for i in range(TAIL):
            for rr in range(8):
                pltpu.async_copy(
                    vb.at[1, i, rr],
                    out_hbm.at[pl.ds((base + NR_FULL * RND + i) * 512
                                     + rr * D, D)],
                    osem)
        n = TAIL * 8 * D
        pltpu.make_async_copy(out_hbm.at[pl.ds(0, n)],
                              vdrain.at[pl.ds(0, n)], osem).wait()

        @pl.when(wid == NW - 1)
        def _():
            gbase = NW * SPW
            pltpu.sync_copy(w3_hbm.at[pl.ds(gbase, EXTRA)],
                            vb.at[0, pl.ds(0, EXTRA)])
            for i in range(EXTRA):
                for rr in range(8):
                    pltpu.async_copy(
                        vb.at[0, i, rr],
                        out_hbm.at[pl.ds((gbase + i) * 512 + rr * D, D)],
                        osem)
            n2 = EXTRA * 8 * D
            pltpu.make_async_copy(out_hbm.at[pl.ds(0, n2)],
                                  vdrain.at[pl.ds(0, n2)], osem).wait()

    return depad_kernel


def _build_gather():
    mesh = plsc.VectorSubcoreMesh(core_axis_name="c", subcore_axis_name="s")

    @functools.partial(
        pl.kernel,
        mesh=mesh,
        out_type=jax.ShapeDtypeStruct((S, 8, NW, 8, 128), jnp.float32),
        compiler_params=pltpu.CompilerParams(use_tc_tiling_on_sc=False,
                                             needs_layout_passes=False),
        scratch_types=[
            pltpu.VMEM((BW, S + 1), jnp.int32),
            pltpu.VMEM((2, BW), jnp.int32),
            pltpu.VMEM((2, BW, D), jnp.float32),
            pltpu.VMEM((2, D, 128), jnp.float32),
        ] + [pltpu.SemaphoreType.DMA] * 4,
    )
    def gather_kernel(table_hbm, idx_hbm, out_hbm, idx_v, list_v, g_v, st_v,
                      gsem0, gsem1, osem0, osem1):
        wid = lax.axis_index("s") * NC + lax.axis_index("c")
        gsem = (gsem0, gsem1)
        osem = (osem0, osem1)
        pltpu.sync_copy(idx_hbm.at[pl.ds(wid * BW, BW)],
                        idx_v.at[pl.ds(0, BW), pl.ds(0, S)])

        jv = lax.iota(jnp.int32, L)

        def fire(s, p):
            sv = jnp.full((L,), 0, jnp.int32) + s
            for k in range(BW // L):
                rv = jv + (L * k)
                tv = plsc.load_gather(idx_v, [rv, sv])
                list_v[p, pl.ds(L * k, L)] = tv
            pltpu.async_copy(table_hbm.at[list_v.at[p]], g_v.at[p], gsem[p])

        def wait_g(p):
            pltpu.make_async_copy(
                table_hbm.at[list_v.at[p]], g_v.at[p], gsem[p]).wait()

        def transpose(p):
            g2 = g_v.at[p]
            st2 = st_v.at[p]

            @plsc.parallel_loop(0, D, step=8, unroll=4)
            def _(c0):
                for cc in range(8):
                    dv = (jv + (c0 + cc)) & (D - 1)
                    for k in range(BW // L):
                        rv = jv + (L * k)
                        v = plsc.load_gather(g2, [rv, dv])
                        plsc.store_scatter(st2, [dv, rv], v)

        def fire_out(s, p):
            for dt in range(8):
                pltpu.async_copy(st_v.at[p, pl.ds(8 * dt, 8)],
                                 out_hbm.at[s, dt, wid], osem[p])

        def wait_out(p):
            for dt in range(8):
                pltpu.make_async_copy(out_hbm.at[0, dt, 0],
                                      st_v.at[p, pl.ds(8 * dt, 8)],
                                      osem[p]).wait()

        fire(0, 0)

        @pl.loop(0, S, step=2)
        def _(s0):
            for b in range(2):
                s = s0 + b
                p = b

                @pl.when(s < S - 1)
                def _():
                    fire(s + 1, 1 - p)

                wait_g(p)

                @pl.when(s >= 2)
                def _():
                    wait_out(p)

                transpose(p)
                fire_out(s, p)

        wait_out(0)
        wait_out(1)

    return gather_kernel


def kernel(x, weight):
    wlin = _build_depad()(weight.reshape(SLABS, 8, D))
    o5 = _build_gather()(wlin.reshape(SLABS * 8, D), x.astype(jnp.int32))
    out = o5.transpose(0, 1, 3, 2, 4).reshape(S, D, B_TOK)
    return out.transpose(2, 0, 1)
